# Initial kernel scaffold; baseline (speedup 1.0000x reference)
import jax
import jax.numpy as jnp
from jax.experimental import pallas as pl


def kernel(x, labels):
    raise NotImplementedError("write your pallas kernel here")


# speedup vs baseline: 2.9012x; 2.9012x over previous
import jax
import jax.numpy as jnp
from jax import lax
from jax.experimental import pallas as pl
from jax.experimental.pallas import tpu as pltpu
from jax.experimental.pallas import tpu_sc as plsc

_WINDOW = 1000
_STRIDE = 200
_ROWS = 16


def _body(num_new, x_hbm, labels_hbm, out_hbm, lbl_hbm,
          buf0, buf1, lblv, lblb, gsem, ssem):
    n = lax.axis_index("c") * 16 + lax.axis_index("s")
    channels = x_hbm.shape[1]
    h = x_hbm.shape[2]
    groups_per_c = h // _ROWS
    num_groups = channels * groups_per_c

    pltpu.sync_copy(labels_hbm, lblv)
    num_samples = x_hbm.shape[0]
    chunk_id = jnp.zeros((16,), jnp.int32) + (n // 16)
    chunk = jnp.zeros((16,), jnp.float32)
    for i in range(num_samples // 16):
        chunk = jnp.where(chunk_id == i, lblv[pl.ds(16 * i, 16)], chunk)
    lanes = lax.iota(jnp.int32, 16)
    val = jnp.sum(jnp.where(lanes == (n % 16), chunk, 0.0))
    lblb[...] = jnp.zeros((16,), jnp.float32) + val
    pltpu.sync_copy(lblb, lbl_hbm.at[pl.ds(n * num_new, num_new)])

    bufs = (buf0, buf1)

    def start_gather(g):
        c, h0 = g // groups_per_c, (g % groups_per_c) * _ROWS
        return pltpu.async_copy(x_hbm.at[n, c, pl.ds(h0, _ROWS), :],
                                bufs[g % 2], gsem)

    pending = start_gather(0)
    for g in range(num_groups):
        pending.wait()
        if g + 1 < num_groups:
            pending = start_gather(g + 1)
        buf = bufs[g % 2]
        c, h0 = g // groups_per_c, (g % groups_per_c) * _ROWS
        scats = []
        for j in range(num_new):
            s = n * num_new + j
            scats.append(pltpu.async_copy(
                buf.at[:, pl.ds(j * _STRIDE, _WINDOW)],
                out_hbm.at[s, c, pl.ds(h0, _ROWS), :], ssem))
        for d in scats:
            d.wait()


def kernel(x, labels):
    num_samples, channels, h, w = x.shape
    num_new = (w - _WINDOW) // _STRIDE + 1
    mesh = plsc.VectorSubcoreMesh(core_axis_name="c", subcore_axis_name="s")
    out_type = (
        jax.ShapeDtypeStruct((num_samples * num_new, channels, h, _WINDOW),
                             x.dtype),
        jax.ShapeDtypeStruct((num_samples * num_new,), labels.dtype),
    )
    import functools
    f = pl.kernel(
        functools.partial(_body, num_new),
        out_type=out_type,
        mesh=mesh,
        compiler_params=pltpu.CompilerParams(use_tc_tiling_on_sc=False,
                                             needs_layout_passes=False),
        scratch_types=[
            pltpu.VMEM((_ROWS, w), jnp.float32),
            pltpu.VMEM((_ROWS, w), jnp.float32),
            pltpu.VMEM((num_samples,), jnp.float32),
            pltpu.VMEM((16,), jnp.float32),
            pltpu.SemaphoreType.DMA,
            pltpu.SemaphoreType.DMA,
        ],
    )
    return f(x, labels)
